# baseline (speedup 1.0000x reference)
import jax
import jax.numpy as jnp
from jax import lax
from jax.experimental import pallas as pl
from jax.experimental.pallas import tpu as pltpu
from jax.experimental.pallas import tpu_sc as plsc

LATENT = 32
NC = 2
NS = 16
NW = NC * NS
L = 16
CHUNK = 128
SUPER = 128
GPC = CHUNK // L


def _gmf_body(uidx_hbm, iidx_hbm, utab_hbm, itab_hbm, w_hbm, b_hbm, out_hbm,
              uidx_v, iidx_v, usid_v, isid_v, ubuf_v, ibuf_v, w_v, b_v,
              out_v, sem):
    wid = lax.axis_index("s") * NC + lax.axis_index("c")
    bpw = out_v.shape[0]
    nch = uidx_v.shape[0]
    base = wid * bpw

    pltpu.sync_copy(uidx_hbm.at[wid], uidx_v)
    pltpu.sync_copy(iidx_hbm.at[wid], iidx_v)
    pltpu.sync_copy(w_hbm, w_v)
    pltpu.sync_copy(b_hbm, b_v)

    for j in range(nch):
        for v in range(GPC):
            uvec = uidx_v[j, pl.ds(v * L, L)]
            usid_v[j, pl.ds(v * L, L)] = lax.shift_right_logical(uvec, 2)
            ivec = iidx_v[j, pl.ds(v * L, L)]
            isid_v[j, pl.ds(v * L, L)] = lax.shift_right_logical(ivec, 2)

    b_vec = b_v[...]
    w_lo = w_v[pl.ds(0, L)]
    w_hi = w_v[pl.ds(L, L)]
    w_scalars = [w_lo[d] for d in range(L)] + [w_hi[d] for d in range(L)]
    lanes = lax.iota(jnp.int32, L)
    three = jnp.full((L,), 3, jnp.int32)

    for j in range(nch):
        hu = pltpu.async_copy(utab_hbm.at[usid_v.at[j]], ubuf_v, sem)
        hi = pltpu.async_copy(itab_hbm.at[isid_v.at[j]], ibuf_v, sem)
        hu.wait()
        hi.wait()

        def group(g, carry):
            lrow = g * L + lanes
            uvec = uidx_v[j, pl.ds(pl.multiple_of(g * L, L), L)]
            ivec = iidx_v[j, pl.ds(pl.multiple_of(g * L, L), L)]
            ucol = lax.shift_left(uvec & three, 5)
            icol = lax.shift_left(ivec & three, 5)
            acc = b_vec
            for d in range(LATENT):
                uc = plsc.load_gather(ubuf_v, [lrow, ucol + d])
                ic = plsc.load_gather(ibuf_v, [lrow, icol + d])
                acc = acc + uc * ic * w_scalars[d]
            off = pl.multiple_of(j * CHUNK + g * L, L)
            out_v[pl.ds(off, L)] = 1.0 / (1.0 + jnp.exp(-acc))
            return carry

        lax.fori_loop(0, GPC, group, 0)

    pltpu.sync_copy(out_v, out_hbm.at[pl.ds(base, bpw)])


def kernel(user_indices, item_indices, emb_user_gmf, emb_item_gmf, W_aff, b_aff):
    batch = user_indices.shape[0]
    bpw = batch // NW
    nch = bpw // CHUNK
    nsup = emb_user_gmf.shape[0] * LATENT // SUPER
    uidx = user_indices.astype(jnp.int32).reshape(NW, nch, CHUNK)
    iidx = item_indices.astype(jnp.int32).reshape(NW, nch, CHUNK)
    utab = emb_user_gmf.reshape(nsup, SUPER)
    itab = emb_item_gmf.reshape(nsup, SUPER)
    w = W_aff.reshape(LATENT).astype(jnp.float32)
    b = jnp.broadcast_to(b_aff.reshape(()), (L,)).astype(jnp.float32)

    fn = pl.kernel(
        _gmf_body,
        mesh=plsc.VectorSubcoreMesh(core_axis_name="c", subcore_axis_name="s"),
        compiler_params=pltpu.CompilerParams(needs_layout_passes=False),
        out_type=jax.ShapeDtypeStruct((batch,), jnp.float32),
        scratch_types=[
            pltpu.VMEM((nch, CHUNK), jnp.int32),
            pltpu.VMEM((nch, CHUNK), jnp.int32),
            pltpu.VMEM((nch, CHUNK), jnp.int32),
            pltpu.VMEM((nch, CHUNK), jnp.int32),
            pltpu.VMEM((CHUNK, SUPER), jnp.float32),
            pltpu.VMEM((CHUNK, SUPER), jnp.float32),
            pltpu.VMEM((LATENT,), jnp.float32),
            pltpu.VMEM((L,), jnp.float32),
            pltpu.VMEM((bpw,), jnp.float32),
            pltpu.SemaphoreType.DMA,
        ],
    )
    out = fn(uidx, iidx, utab, itab, w, b)
    return out.reshape(batch, 1)


# speedup vs baseline: 1.0010x; 1.0010x over previous
import jax
import jax.numpy as jnp
from jax import lax
from jax.experimental import pallas as pl
from jax.experimental.pallas import tpu as pltpu
from jax.experimental.pallas import tpu_sc as plsc

LATENT = 32
NC = 2
NS = 16
NW = NC * NS
L = 16
CHUNK = 128
SUPER = 128
GPC = CHUNK // L


def _gmf_body(uidx_hbm, iidx_hbm, utab_hbm, itab_hbm, w_hbm, b_hbm, out_hbm,
              uidx_v, iidx_v, usid_v, isid_v, ubuf_v, ibuf_v, w_v, b_v,
              out_v, sem):
    wid = lax.axis_index("s") * NC + lax.axis_index("c")
    bpw = out_v.shape[0]
    nch = uidx_v.shape[0]
    base = wid * bpw

    pltpu.sync_copy(uidx_hbm.at[wid], uidx_v)
    pltpu.sync_copy(iidx_hbm.at[wid], iidx_v)
    pltpu.sync_copy(w_hbm, w_v)
    pltpu.sync_copy(b_hbm, b_v)

    for j in range(nch):
        for v in range(GPC):
            uvec = uidx_v[j, pl.ds(v * L, L)]
            usid_v[j, pl.ds(v * L, L)] = lax.shift_right_logical(uvec, 2)
            ivec = iidx_v[j, pl.ds(v * L, L)]
            isid_v[j, pl.ds(v * L, L)] = lax.shift_right_logical(ivec, 2)

    b_vec = b_v[...]
    w_lo = w_v[pl.ds(0, L)]
    w_hi = w_v[pl.ds(L, L)]
    w_scalars = [w_lo[d] for d in range(L)] + [w_hi[d] for d in range(L)]
    lanes = lax.iota(jnp.int32, L)
    three = jnp.full((L,), 3, jnp.int32)

    for j in range(nch):
        hu = pltpu.async_copy(utab_hbm.at[usid_v.at[j]], ubuf_v, sem)
        hi = pltpu.async_copy(itab_hbm.at[isid_v.at[j]], ibuf_v, sem)
        hu.wait()
        hi.wait()

        def group(g, carry):
            lrow = g * L + lanes
            uvec = uidx_v[j, pl.ds(pl.multiple_of(g * L, L), L)]
            ivec = iidx_v[j, pl.ds(pl.multiple_of(g * L, L), L)]
            ucol = lax.shift_left(uvec & three, 5)
            icol = lax.shift_left(ivec & three, 5)
            acc = b_vec
            for d in range(LATENT):
                uc = plsc.load_gather(ubuf_v, [lrow, ucol + d])
                ic = plsc.load_gather(ibuf_v, [lrow, icol + d])
                acc = acc + uc * ic * w_scalars[d]
            off = pl.multiple_of(j * CHUNK + g * L, L)
            out_v[pl.ds(off, L)] = 1.0 / (1.0 + jnp.exp(-acc))
            return carry

        lax.fori_loop(0, GPC, group, 0)

    pltpu.sync_copy(out_v, out_hbm.at[pl.ds(base, bpw)])


def kernel(user_indices, item_indices, emb_user_gmf, emb_item_gmf, W_aff, b_aff):
    batch = user_indices.shape[0]
    bpw = batch // NW
    nch = bpw // CHUNK
    nsup = emb_user_gmf.shape[0] * LATENT // SUPER
    uidx = user_indices.astype(jnp.int32).reshape(NW, nch, CHUNK)
    iidx = item_indices.astype(jnp.int32).reshape(NW, nch, CHUNK)
    utab = emb_user_gmf.reshape(nsup, SUPER)
    itab = emb_item_gmf.reshape(nsup, SUPER)
    w = W_aff.reshape(LATENT).astype(jnp.float32)
    b = jnp.broadcast_to(b_aff.reshape(()), (L,)).astype(jnp.float32)

    fn = pl.kernel(
        _gmf_body,
        mesh=plsc.VectorSubcoreMesh(core_axis_name="c", subcore_axis_name="s"),
        compiler_params=pltpu.CompilerParams(
            needs_layout_passes=False, use_tc_tiling_on_sc=True),
        out_type=jax.ShapeDtypeStruct((batch,), jnp.float32),
        scratch_types=[
            pltpu.VMEM((nch, CHUNK), jnp.int32),
            pltpu.VMEM((nch, CHUNK), jnp.int32),
            pltpu.VMEM((nch, CHUNK), jnp.int32),
            pltpu.VMEM((nch, CHUNK), jnp.int32),
            pltpu.VMEM((CHUNK, SUPER), jnp.float32),
            pltpu.VMEM((CHUNK, SUPER), jnp.float32),
            pltpu.VMEM((LATENT,), jnp.float32),
            pltpu.VMEM((L,), jnp.float32),
            pltpu.VMEM((bpw,), jnp.float32),
            pltpu.SemaphoreType.DMA,
        ],
    )
    out = fn(uidx, iidx, utab, itab, w, b)
    return out.reshape(batch, 1)
